# baseline (speedup 1.0000x reference)
import functools

import jax
import jax.numpy as jnp
from jax import lax
from jax.experimental import pallas as pl
from jax.experimental.pallas import tpu as pltpu
from jax.experimental.pallas import tpu_sc as plsc

_BB = 256


def _gather_rows(idx_t, table):
    seq, batch = idx_t.shape
    dim = table.shape[1]
    wins_per_row = batch // _BB
    num_tiles = 32
    wins_per_tile = (seq * wins_per_row) // num_tiles
    mesh = plsc.VectorSubcoreMesh(core_axis_name="c", subcore_axis_name="s")

    @functools.partial(
        pl.kernel,
        out_type=jax.ShapeDtypeStruct((seq, batch, dim), table.dtype),
        mesh=mesh,
        scratch_types=[
            pltpu.VMEM((_BB, dim), table.dtype),
            pltpu.VMEM((_BB, dim), table.dtype),
            pltpu.VMEM((_BB,), jnp.int32),
            pltpu.VMEM((_BB,), jnp.int32),
            pltpu.SemaphoreType.DMA,
            pltpu.SemaphoreType.DMA,
            pltpu.SemaphoreType.DMA,
            pltpu.SemaphoreType.DMA,
        ],
    )
    def gather_kernel(table_hbm, idx_hbm, out_hbm, buf0, buf1, idx0, idx1,
                      gsem0, gsem1, wsem0, wsem1):
        wid = lax.axis_index("s") * 2 + lax.axis_index("c")
        base = wid * wins_per_tile
        bufs = (buf0, buf1)
        idxs = (idx0, idx1)
        gsems = (gsem0, gsem1)
        wsems = (wsem0, wsem1)

        def win(t):
            w = base + t
            return w // wins_per_row, (w % wins_per_row) * _BB

        def start_gather(t):
            slot = t % 2
            s, b = win(t)
            pltpu.sync_copy(idx_hbm.at[s, pl.ds(b, _BB)], idxs[slot])
            return pltpu.async_copy(
                table_hbm.at[idxs[slot]], bufs[slot], gsems[slot]
            )

        gathers = [None, None]
        writes = [None, None]
        gathers[0] = start_gather(0)
        for t in range(wins_per_tile):
            slot = t % 2
            nxt = (t + 1) % 2
            if t + 1 < wins_per_tile:
                if writes[nxt] is not None:
                    writes[nxt].wait()
                    writes[nxt] = None
                gathers[nxt] = start_gather(t + 1)
            gathers[slot].wait()
            s, b = win(t)
            writes[slot] = pltpu.async_copy(
                bufs[slot], out_hbm.at[s, pl.ds(b, _BB)], wsems[slot]
            )
        for wcopy in writes:
            if wcopy is not None:
                wcopy.wait()

    return gather_kernel(table, idx_t)


def kernel(input_ids, table):
    idx_t = input_ids.astype(jnp.int32).T
    out_t = _gather_rows(idx_t, table)
    return out_t.transpose(1, 0, 2)


# speedup vs baseline: 1.0027x; 1.0027x over previous
import functools

import jax
import jax.numpy as jnp
from jax import lax
from jax.experimental import pallas as pl
from jax.experimental.pallas import tpu as pltpu
from jax.experimental.pallas import tpu_sc as plsc

_BB = 256


def _gather_rows(idx_t, table):
    seq, batch = idx_t.shape
    dim = table.shape[1]
    wins_per_row = batch // _BB
    num_tiles = 32
    wins_per_tile = (seq * wins_per_row) // num_tiles
    mesh = plsc.VectorSubcoreMesh(core_axis_name="c", subcore_axis_name="s")

    @functools.partial(
        pl.kernel,
        out_type=jax.ShapeDtypeStruct((seq, batch, dim), table.dtype),
        mesh=mesh,
        scratch_types=[
            pltpu.VMEM((_BB, dim), table.dtype),
            pltpu.VMEM((_BB, dim), table.dtype),
            pltpu.VMEM((_BB,), jnp.int32),
            pltpu.VMEM((_BB,), jnp.int32),
            pltpu.VMEM((_BB,), jnp.int32),
            pltpu.SemaphoreType.DMA,
            pltpu.SemaphoreType.DMA,
            pltpu.SemaphoreType.DMA,
            pltpu.SemaphoreType.DMA,
            pltpu.SemaphoreType.DMA,
            pltpu.SemaphoreType.DMA,
            pltpu.SemaphoreType.DMA,
        ],
    )
    def gather_kernel(table_hbm, idx_hbm, out_hbm, buf0, buf1,
                      idx0, idx1, idx2, gsem0, gsem1, wsem0, wsem1,
                      isem0, isem1, isem2):
        wid = lax.axis_index("s") * 2 + lax.axis_index("c")
        base = wid * wins_per_tile
        bufs = (buf0, buf1)
        idxs = (idx0, idx1, idx2)
        gsems = (gsem0, gsem1)
        wsems = (wsem0, wsem1)
        isems = (isem0, isem1, isem2)

        def win(t):
            w = base + t
            return w // wins_per_row, (w % wins_per_row) * _BB

        def start_idx_load(t):
            k = t % 3
            s, b = win(t)
            return pltpu.async_copy(
                idx_hbm.at[s, pl.ds(b, _BB)], idxs[k], isems[k]
            )

        def start_gather(t, idx_loads):
            slot = t % 2
            idx_loads[t % 3].wait()
            return pltpu.async_copy(
                table_hbm.at[idxs[t % 3]], bufs[slot], gsems[slot]
            )

        idx_loads = [None, None, None]
        for t in range(min(3, wins_per_tile)):
            idx_loads[t] = start_idx_load(t)
        gathers = [None, None]
        writes = [None, None]
        gathers[0] = start_gather(0, idx_loads)
        for t in range(wins_per_tile):
            slot = t % 2
            nxt = (t + 1) % 2
            if t + 1 < wins_per_tile:
                if writes[nxt] is not None:
                    writes[nxt].wait()
                    writes[nxt] = None
                gathers[nxt] = start_gather(t + 1, idx_loads)
            gathers[slot].wait()
            if t + 3 < wins_per_tile:
                idx_loads[t % 3] = start_idx_load(t + 3)
            s, b = win(t)
            writes[slot] = pltpu.async_copy(
                bufs[slot], out_hbm.at[s, pl.ds(b, _BB)], wsems[slot]
            )
        for wcopy in writes:
            if wcopy is not None:
                wcopy.wait()

    return gather_kernel(table, idx_t)


def kernel(input_ids, table):
    idx_t = input_ids.astype(jnp.int32).T
    out_t = _gather_rows(idx_t, table)
    return out_t.transpose(1, 0, 2)


# speedup vs baseline: 1.0089x; 1.0062x over previous
import functools

import jax
import jax.numpy as jnp
from jax import lax
from jax.experimental import pallas as pl
from jax.experimental.pallas import tpu as pltpu
from jax.experimental.pallas import tpu_sc as plsc

_BB = 256
_DEPTH = 3


def _gather_rows(idx_t, table):
    seq, batch = idx_t.shape
    dim = table.shape[1]
    wins_per_row = batch // _BB
    num_tiles = 32
    wins_per_tile = (seq * wins_per_row) // num_tiles
    mesh = plsc.VectorSubcoreMesh(core_axis_name="c", subcore_axis_name="s")

    @functools.partial(
        pl.kernel,
        out_type=jax.ShapeDtypeStruct((seq, batch, dim), table.dtype),
        mesh=mesh,
        scratch_types=(
            [pltpu.VMEM((_BB, dim), table.dtype)] * _DEPTH
            + [pltpu.VMEM((_BB,), jnp.int32)] * _DEPTH
            + [pltpu.SemaphoreType.DMA] * (3 * _DEPTH)
        ),
    )
    def gather_kernel(table_hbm, idx_hbm, out_hbm, *scratch):
        bufs = scratch[:_DEPTH]
        idxs = scratch[_DEPTH:2 * _DEPTH]
        gsems = scratch[2 * _DEPTH:3 * _DEPTH]
        wsems = scratch[3 * _DEPTH:4 * _DEPTH]
        isems = scratch[4 * _DEPTH:5 * _DEPTH]
        wid = lax.axis_index("s") * 2 + lax.axis_index("c")
        base = wid * wins_per_tile

        def win(t):
            w = base + t
            return w // wins_per_row, (w % wins_per_row) * _BB

        def start_idx_load(t):
            k = t % _DEPTH
            s, b = win(t)
            return pltpu.async_copy(
                idx_hbm.at[s, pl.ds(b, _BB)], idxs[k], isems[k]
            )

        def start_gather(t, idx_loads):
            k = t % _DEPTH
            idx_loads[k].wait()
            return pltpu.async_copy(
                table_hbm.at[idxs[k]], bufs[k], gsems[k]
            )

        idx_loads = [None] * _DEPTH
        for t in range(min(_DEPTH, wins_per_tile)):
            idx_loads[t] = start_idx_load(t)
        gathers = [None] * _DEPTH
        writes = [None] * _DEPTH
        for t in range(min(_DEPTH - 1, wins_per_tile)):
            gathers[t] = start_gather(t, idx_loads)
        for t in range(wins_per_tile):
            k = t % _DEPTH
            ahead = t + _DEPTH - 1
            if ahead < wins_per_tile:
                ka = ahead % _DEPTH
                if writes[ka] is not None:
                    writes[ka].wait()
                    writes[ka] = None
                gathers[ka] = start_gather(ahead, idx_loads)
            gathers[k].wait()
            if t + _DEPTH < wins_per_tile:
                idx_loads[k] = start_idx_load(t + _DEPTH)
            s, b = win(t)
            writes[k] = pltpu.async_copy(
                bufs[k], out_hbm.at[s, pl.ds(b, _BB)], wsems[k]
            )
        for wcopy in writes:
            if wcopy is not None:
                wcopy.wait()

    return gather_kernel(table, idx_t)


def kernel(input_ids, table):
    idx_t = input_ids.astype(jnp.int32).T
    out_t = _gather_rows(idx_t, table)
    return out_t.transpose(1, 0, 2)
